# Initial kernel scaffold; baseline (speedup 1.0000x reference)
import jax
import jax.numpy as jnp
from jax.experimental import pallas as pl


def kernel(x, weight):
    raise NotImplementedError("write your pallas kernel here")


# speedup vs baseline: 1.6843x; 1.6843x over previous
import functools

import jax
import jax.numpy as jnp
from jax import lax
from jax.experimental import pallas as pl
from jax.experimental.pallas import tpu as pltpu
from jax.experimental.pallas import tpu_sc as plsc

NC, NS = 2, 16
NW = NC * NS
B = 16384 * 50
D = 64
ROWS_PER_W = B // NW
CHUNK = 128
NCHUNK = ROWS_PER_W // CHUNK

_MESH = plsc.VectorSubcoreMesh(
    core_axis_name="c", subcore_axis_name="s", num_cores=NC, num_subcores=NS
)


@functools.partial(
    pl.kernel,
    out_type=jax.ShapeDtypeStruct((B, D), jnp.float32),
    mesh=_MESH,
    scratch_types=[
        pltpu.VMEM((NCHUNK, CHUNK), jnp.int32),
        pltpu.VMEM((CHUNK, D), jnp.float32),
        pltpu.SemaphoreType.DMA,
    ],
    compiler_params=pltpu.CompilerParams(use_tc_tiling_on_sc=False),
)
def _embed_gather(table_hbm, idx_hbm, out_hbm, idx_v, rows_v, sem):
    wid = lax.axis_index("s") * NC + lax.axis_index("c")
    base = wid * ROWS_PER_W
    pltpu.sync_copy(idx_hbm.at[wid], idx_v)

    def body(j, carry):
        pltpu.async_copy(table_hbm.at[idx_v.at[j]], rows_v, sem).wait()
        pltpu.sync_copy(rows_v, out_hbm.at[pl.ds(base + j * CHUNK, CHUNK)])
        return carry

    lax.fori_loop(0, NCHUNK, body, 0)


def kernel(x, weight):
    idx = x.reshape(NW, NCHUNK, CHUNK)
    out = _embed_gather(weight, idx)
    return out.reshape(x.shape[0], x.shape[1], D)


# speedup vs baseline: 1.8665x; 1.1082x over previous
import functools

import jax
import jax.numpy as jnp
from jax import lax
from jax.experimental import pallas as pl
from jax.experimental.pallas import tpu as pltpu
from jax.experimental.pallas import tpu_sc as plsc

NC, NS = 2, 16
NW = NC * NS
B = 16384 * 50
D = 64
ROWS_PER_W = B // NW
CHUNK = 128
NCHUNK = ROWS_PER_W // CHUNK
NBUF = 4
GPB = 2
SUPER = CHUNK * GPB
NSUP = ROWS_PER_W // SUPER
NOUTER = NSUP // NBUF

_MESH = plsc.VectorSubcoreMesh(
    core_axis_name="c", subcore_axis_name="s", num_cores=NC, num_subcores=NS
)


@functools.partial(
    pl.kernel,
    out_type=jax.ShapeDtypeStruct((B, D), jnp.float32),
    mesh=_MESH,
    scratch_types=[
        pltpu.VMEM((NCHUNK, CHUNK), jnp.int32),
        pltpu.VMEM((NBUF, SUPER, D), jnp.float32),
        pltpu.SemaphoreType.DMA,
        pltpu.SemaphoreType.DMA,
        pltpu.SemaphoreType.DMA,
        pltpu.SemaphoreType.DMA,
        pltpu.SemaphoreType.DMA,
        pltpu.SemaphoreType.DMA,
        pltpu.SemaphoreType.DMA,
        pltpu.SemaphoreType.DMA,
    ],
    compiler_params=pltpu.CompilerParams(use_tc_tiling_on_sc=False),
)
def _embed_gather(table_hbm, idx_hbm, out_hbm, idx_v, rows_v,
                  g0, g1, g2, g3, o0, o1, o2, o3):
    gsem = (g0, g1, g2, g3)
    osem = (o0, o1, o2, o3)
    wid = lax.axis_index("s") * NC + lax.axis_index("c")
    base = wid * ROWS_PER_W
    pltpu.sync_copy(idx_hbm.at[wid], idx_v)

    def fire_gathers(g, b):
        for q in range(GPB):
            pltpu.async_copy(
                table_hbm.at[idx_v.at[g * GPB + q]],
                rows_v.at[b, pl.ds(q * CHUNK, CHUNK)],
                gsem[b],
            )

    def drain_gathers(b):
        for q in range(GPB):
            pltpu.make_async_copy(
                table_hbm.at[pl.ds(0, CHUNK)],
                rows_v.at[b, pl.ds(q * CHUNK, CHUNK)],
                gsem[b],
            ).wait()

    def fire_out(g, b):
        pltpu.async_copy(
            rows_v.at[b],
            out_hbm.at[pl.ds(base + g * SUPER, SUPER)],
            osem[b],
        )

    def drain_out(b):
        pltpu.make_async_copy(
            table_hbm.at[pl.ds(0, SUPER)],
            rows_v.at[b],
            osem[b],
        ).wait()

    for b in range(NBUF):
        fire_gathers(b, b)

    def outer(p, carry):
        for b in range(NBUF):
            g = p * NBUF + b
            drain_gathers(b)
            fire_out(g, b)
        for b in range(NBUF):
            g_next = (p + 1) * NBUF + b
            drain_out(b)

            @pl.when(g_next < NSUP)
            def _():
                fire_gathers(g_next, b)

        return carry

    lax.fori_loop(0, NOUTER, outer, 0)


def kernel(x, weight):
    idx = x.reshape(NW, NCHUNK, CHUNK)
    out = _embed_gather(weight, idx)
    return out.reshape(x.shape[0], x.shape[1], D)
